# Initial kernel scaffold; baseline (speedup 1.0000x reference)
import jax
import jax.numpy as jnp
from jax.experimental import pallas as pl


def kernel(z, codebook):
    raise NotImplementedError("write your pallas kernel here")


# speedup vs baseline: 8.6772x; 8.6772x over previous
import jax
import jax.numpy as jnp
from jax.experimental import pallas as pl
from jax.experimental.pallas import tpu as pltpu

_DIM = 14
_N_E = 16384
_N_TOK = 4096
_TEMP = 0.01
_EPS = 1e-5
_BETA = 0.25
_ENTROPY_LOSS_RATIO = 0.1


def _lfq_body(z2d_ref, zt_ref, sign_ref, idx_ref, se_ref, ae_ref, aux_ref,
              cl_ref):
    z = z2d_ref[...]
    zt = zt_ref[...]

    pos = z > 0.0
    sgn = jnp.where(pos, 1.0, -1.0).astype(jnp.float32)
    sign_ref[...] = sgn

    lane_d = jax.lax.broadcasted_iota(jnp.int32, (_N_TOK, _DIM), 1)
    w = jnp.left_shift(jnp.int32(1), lane_d)
    idx_ref[...] = jnp.sum(jnp.where(pos, w, 0), axis=1, keepdims=True)

    scale = jnp.float32(4.0 / _TEMP)
    x = jnp.abs(z) * scale
    u = jnp.exp(-x)
    inv = 1.0 / (1.0 + u)
    big = inv
    small = u * inv

    hb = jnp.log1p(u) + x * small
    se = jnp.sum(hb) / jnp.float32(_N_TOK)

    cl = jnp.sum((sgn - z) ** 2) / jnp.float32(_N_TOK * _DIM)

    lane_l = jax.lax.broadcasted_iota(jnp.int32, (_N_TOK, 128), 1)
    lo = None
    for d in range(7):
        lane_bit = jnp.bitwise_and(jnp.right_shift(lane_l, d), 1) == 1
        match = lane_bit == pos[:, d:d + 1]
        f = jnp.where(match, big[:, d:d + 1], small[:, d:d + 1])
        lo = f if lo is None else lo * f

    xt = jnp.abs(zt) * scale
    ut = jnp.exp(-xt)
    invt = 1.0 / (1.0 + ut)
    bigt = invt
    smallt = ut * invt
    post = zt > 0.0
    sub_c = jax.lax.broadcasted_iota(jnp.int32, (128, _N_TOK), 0)
    hit = None
    for d in range(7):
        sub_bit = jnp.bitwise_and(jnp.right_shift(sub_c, d), 1) == 1
        match = sub_bit == post[d + 7:d + 8, :]
        f = jnp.where(match, bigt[d + 7:d + 8, :], smallt[d + 7:d + 8, :])
        hit = f if hit is None else hit * f

    acc = jax.lax.dot_general(
        hit, lo, (((1,), (0,)), ((), ())),
        preferred_element_type=jnp.float32,
        precision=jax.lax.Precision.HIGHEST)
    q = acc * jnp.float32(1.0 / _N_TOK)
    ae = -jnp.sum(q * jnp.log(q + jnp.float32(_EPS)))

    se_ref[0, 0] = se
    ae_ref[0, 0] = ae
    aux_ref[0, 0] = jnp.float32(_ENTROPY_LOSS_RATIO) * (se - ae)
    cl_ref[0, 0] = jnp.float32(_BETA) * cl


def kernel(z, codebook):
    del codebook
    b, d, h, w = z.shape
    z2d = jnp.transpose(z, (0, 2, 3, 1)).reshape(b * h * w, d)
    zt = z.reshape(b, d, h * w).transpose(1, 0, 2).reshape(d, b * h * w)

    smem_scalar = pl.BlockSpec(memory_space=pltpu.SMEM)
    sign2d, idx2d, se, ae, aux, cl = pl.pallas_call(
        _lfq_body,
        out_shape=(
            jax.ShapeDtypeStruct((_N_TOK, _DIM), jnp.float32),
            jax.ShapeDtypeStruct((_N_TOK, 1), jnp.int32),
            jax.ShapeDtypeStruct((1, 1), jnp.float32),
            jax.ShapeDtypeStruct((1, 1), jnp.float32),
            jax.ShapeDtypeStruct((1, 1), jnp.float32),
            jax.ShapeDtypeStruct((1, 1), jnp.float32),
        ),
        out_specs=(
            pl.BlockSpec(memory_space=pltpu.VMEM),
            pl.BlockSpec(memory_space=pltpu.VMEM),
            smem_scalar, smem_scalar, smem_scalar, smem_scalar,
        ),
    )(z2d, zt)

    q = sign2d.reshape(b, h, w, d).transpose(0, 3, 1, 2)
    indices_flat = idx2d.reshape(-1)
    return (q,
            (se[0, 0], ae[0, 0], aux[0, 0], cl[0, 0]),
            indices_flat)


# speedup vs baseline: 16.6530x; 1.9192x over previous
import jax
import jax.numpy as jnp
from jax.experimental import pallas as pl
from jax.experimental.pallas import tpu as pltpu

_DIM = 14
_N_E = 16384
_TEMP = 0.01
_EPS = 1e-5
_BETA = 0.25
_ENTROPY_LOSS_RATIO = 0.1


def _lfq_body(z_ref, sign_ref, idx_ref, se_ref, ae_ref, aux_ref, cl_ref):
    z = z_ref[...]
    bsz, _, npix = z.shape
    n_tok = bsz * npix

    pos = z > 0.0
    sgn = jnp.where(pos, 1.0, -1.0).astype(jnp.float32)
    sign_ref[...] = sgn

    d_iota = jax.lax.broadcasted_iota(jnp.int32, z.shape, 1)
    w = jnp.left_shift(jnp.int32(1), d_iota)
    idx_ref[...] = jnp.sum(jnp.where(pos, w, 0), axis=1)

    scale = jnp.float32(4.0 / _TEMP)
    x = jnp.abs(z) * scale
    u = jnp.exp(-x)
    inv = 1.0 / (1.0 + u)
    big = inv
    small = u * inv

    hb = jnp.log1p(u) + x * small
    se = jnp.sum(hb) / jnp.float32(n_tok)

    cl = jnp.sum((sgn - z) ** 2) / jnp.float32(n_tok * _DIM)

    sub_c = jax.lax.broadcasted_iota(jnp.int32, (128, npix), 0)
    acc = jnp.zeros((128, 128), jnp.float32)
    for b in range(bsz):
        pos_b = pos[b]
        big_b = big[b]
        small_b = small[b]
        lot = None
        hit = None
        for d in range(7):
            sub_bit = jnp.bitwise_and(jnp.right_shift(sub_c, d), 1) == 1
            fl = jnp.where(sub_bit == pos_b[d:d + 1, :],
                           big_b[d:d + 1, :], small_b[d:d + 1, :])
            lot = fl if lot is None else lot * fl
            fh = jnp.where(sub_bit == pos_b[d + 7:d + 8, :],
                           big_b[d + 7:d + 8, :], small_b[d + 7:d + 8, :])
            hit = fh if hit is None else hit * fh
        acc = acc + jax.lax.dot_general(
            hit, lot, (((1,), (1,)), ((), ())),
            preferred_element_type=jnp.float32,
            precision=jax.lax.Precision.HIGHEST)

    q = acc * (1.0 / jnp.float32(n_tok))
    ae = -jnp.sum(q * jnp.log(q + jnp.float32(_EPS)))

    se_ref[0, 0] = se
    ae_ref[0, 0] = ae
    aux_ref[0, 0] = jnp.float32(_ENTROPY_LOSS_RATIO) * (se - ae)
    cl_ref[0, 0] = jnp.float32(_BETA) * cl


def kernel(z, codebook):
    del codebook
    b, d, h, w = z.shape
    z3d = z.reshape(b, d, h * w)

    smem_scalar = pl.BlockSpec(memory_space=pltpu.SMEM)
    sign3d, idx2d, se, ae, aux, cl = pl.pallas_call(
        _lfq_body,
        out_shape=(
            jax.ShapeDtypeStruct((b, d, h * w), jnp.float32),
            jax.ShapeDtypeStruct((b, h * w), jnp.int32),
            jax.ShapeDtypeStruct((1, 1), jnp.float32),
            jax.ShapeDtypeStruct((1, 1), jnp.float32),
            jax.ShapeDtypeStruct((1, 1), jnp.float32),
            jax.ShapeDtypeStruct((1, 1), jnp.float32),
        ),
        out_specs=(
            pl.BlockSpec(memory_space=pltpu.VMEM),
            pl.BlockSpec(memory_space=pltpu.VMEM),
            smem_scalar, smem_scalar, smem_scalar, smem_scalar,
        ),
    )(z3d)

    q = sign3d.reshape(b, d, h, w)
    indices_flat = idx2d.reshape(-1)
    return (q,
            (se[0, 0], ae[0, 0], aux[0, 0], cl[0, 0]),
            indices_flat)


# speedup vs baseline: 36.3677x; 2.1838x over previous
import jax
import jax.numpy as jnp
from jax.experimental import pallas as pl
from jax.experimental.pallas import tpu as pltpu

_DIM = 14
_N_E = 16384
_TEMP = 0.01
_EPS = 1e-5
_BETA = 0.25
_ENTROPY_LOSS_RATIO = 0.1


def _kron7(pos_b, big_b, small_b, base, npix):
    sub_a = jax.lax.broadcasted_iota(jnp.int32, (8, npix), 0)
    a = None
    for d in range(3):
        bit = jnp.bitwise_and(jnp.right_shift(sub_a, d), 1) == 1
        dd = base + d
        f = jnp.where(bit == pos_b[dd:dd + 1, :],
                      big_b[dd:dd + 1, :], small_b[dd:dd + 1, :])
        a = f if a is None else a * f
    sub_b = jax.lax.broadcasted_iota(jnp.int32, (16, npix), 0)
    bb = None
    for d in range(4):
        bit = jnp.bitwise_and(jnp.right_shift(sub_b, d), 1) == 1
        dd = base + 3 + d
        f = jnp.where(bit == pos_b[dd:dd + 1, :],
                      big_b[dd:dd + 1, :], small_b[dd:dd + 1, :])
        bb = f if bb is None else bb * f
    return (bb[:, None, :] * a[None, :, :]).reshape(128, npix)


def _lfq_body(z_ref, sign_ref, idx_ref, se_ref, ae_ref, aux_ref, cl_ref):
    z4 = z_ref[...]
    bsz, _, hh, ww = z4.shape
    npix = hh * ww
    n_tok = bsz * npix

    pos4 = z4 > 0.0
    sgn4 = jnp.where(pos4, 1.0, -1.0).astype(jnp.float32)
    sign_ref[...] = sgn4

    z = z4.reshape(bsz, _DIM, npix)
    pos = z > 0.0

    d_iota = jax.lax.broadcasted_iota(jnp.int32, z.shape, 1)
    w = jnp.left_shift(jnp.int32(1), d_iota)
    idx_ref[...] = jnp.sum(jnp.where(pos, w, 0), axis=1)

    scale = jnp.float32(4.0 / _TEMP)
    x = jnp.abs(z) * scale
    u = jnp.exp(-x)
    inv = 1.0 / (1.0 + u)
    big = inv
    small = u * inv

    hb = jnp.log1p(u) + x * small
    se = jnp.sum(hb) / jnp.float32(n_tok)

    cl = jnp.sum((sgn4 - z4) ** 2) / jnp.float32(n_tok * _DIM)

    acc = jnp.zeros((128, 128), jnp.float32)
    for b in range(bsz):
        lot = _kron7(pos[b], big[b], small[b], 0, npix)
        hit = _kron7(pos[b], big[b], small[b], 7, npix)
        acc = acc + jax.lax.dot_general(
            hit, lot, (((1,), (1,)), ((), ())),
            preferred_element_type=jnp.float32,
            precision=jax.lax.Precision.HIGHEST)

    q = acc * (1.0 / jnp.float32(n_tok))
    ae = -jnp.sum(q * jnp.log(q + jnp.float32(_EPS)))

    se_ref[0, 0] = se
    ae_ref[0, 0] = ae
    aux_ref[0, 0] = jnp.float32(_ENTROPY_LOSS_RATIO) * (se - ae)
    cl_ref[0, 0] = jnp.float32(_BETA) * cl


def kernel(z, codebook):
    del codebook
    b, d, h, w = z.shape

    smem_scalar = pl.BlockSpec(memory_space=pltpu.SMEM)
    q, idx2d, se, ae, aux, cl = pl.pallas_call(
        _lfq_body,
        out_shape=(
            jax.ShapeDtypeStruct((b, d, h, w), jnp.float32),
            jax.ShapeDtypeStruct((b, h * w), jnp.int32),
            jax.ShapeDtypeStruct((1, 1), jnp.float32),
            jax.ShapeDtypeStruct((1, 1), jnp.float32),
            jax.ShapeDtypeStruct((1, 1), jnp.float32),
            jax.ShapeDtypeStruct((1, 1), jnp.float32),
        ),
        out_specs=(
            pl.BlockSpec(memory_space=pltpu.VMEM),
            pl.BlockSpec(memory_space=pltpu.VMEM),
            smem_scalar, smem_scalar, smem_scalar, smem_scalar,
        ),
    )(z)

    indices_flat = idx2d.reshape(-1)
    return (q,
            (se[0, 0], ae[0, 0], aux[0, 0], cl[0, 0]),
            indices_flat)


# speedup vs baseline: 43.5877x; 1.1985x over previous
import jax
import jax.numpy as jnp
from jax.experimental import pallas as pl
from jax.experimental.pallas import tpu as pltpu

_DIM = 14
_N_E = 16384
_TEMP = 0.01
_EPS = 1e-5
_BETA = 0.25
_ENTROPY_LOSS_RATIO = 0.1


def _kron7(pos_b, big_b, small_b, base, npix):
    sub_a = jax.lax.broadcasted_iota(jnp.int32, (8, npix), 0)
    a = None
    for d in range(3):
        bit = jnp.bitwise_and(jnp.right_shift(sub_a, d), 1) == 1
        dd = base + d
        f = jnp.where(bit == pos_b[dd:dd + 1, :],
                      big_b[dd:dd + 1, :], small_b[dd:dd + 1, :])
        a = f if a is None else a * f
    sub_b = jax.lax.broadcasted_iota(jnp.int32, (16, npix), 0)
    bb = None
    for d in range(4):
        bit = jnp.bitwise_and(jnp.right_shift(sub_b, d), 1) == 1
        dd = base + 3 + d
        f = jnp.where(bit == pos_b[dd:dd + 1, :],
                      big_b[dd:dd + 1, :], small_b[dd:dd + 1, :])
        bb = f if bb is None else bb * f
    return (bb[:, None, :] * a[None, :, :]).reshape(128, npix)


def _lfq_body(z_ref, sign_ref, idx_ref, se_ref, ae_ref, aux_ref, cl_ref):
    z4 = z_ref[...]
    bsz, _, hh, ww = z4.shape
    npix = hh * ww
    n_tok = bsz * npix

    pos4 = z4 > 0.0
    sgn4 = jnp.where(pos4, 1.0, -1.0).astype(jnp.float32)
    sign_ref[...] = sgn4

    z = z4.reshape(bsz, _DIM, npix)
    pos = z > 0.0

    d_iota = jax.lax.broadcasted_iota(jnp.int32, z.shape, 1)
    w = jnp.left_shift(jnp.int32(1), d_iota)
    idx_ref[...] = jnp.sum(jnp.where(pos, w, 0), axis=1).reshape(-1)

    scale = jnp.float32(4.0 / _TEMP)
    x = jnp.abs(z) * scale
    u = jnp.exp(-x)
    inv = 1.0 / (1.0 + u)
    big = inv
    small = u * inv

    hb = jnp.log1p(u) + x * small
    se = jnp.sum(hb) / jnp.float32(n_tok)

    cl = jnp.sum((sgn4 - z4) ** 2) / jnp.float32(n_tok * _DIM)

    acc = jnp.zeros((128, 128), jnp.float32)
    for b in range(bsz):
        lot = _kron7(pos[b], big[b], small[b], 0, npix)
        hit = _kron7(pos[b], big[b], small[b], 7, npix)
        acc = acc + jax.lax.dot_general(
            hit, lot, (((1,), (1,)), ((), ())),
            preferred_element_type=jnp.float32,
            precision=jax.lax.Precision.HIGHEST)

    q = acc * (1.0 / jnp.float32(n_tok))
    ae = -jnp.sum(q * jnp.log(q + jnp.float32(_EPS)))

    se_ref[...] = se
    ae_ref[...] = ae
    aux_ref[...] = jnp.float32(_ENTROPY_LOSS_RATIO) * (se - ae)
    cl_ref[...] = jnp.float32(_BETA) * cl


def kernel(z, codebook):
    del codebook
    b, d, h, w = z.shape

    smem_scalar = pl.BlockSpec(memory_space=pltpu.SMEM)
    q, indices_flat, se, ae, aux, cl = pl.pallas_call(
        _lfq_body,
        out_shape=(
            jax.ShapeDtypeStruct((b, d, h, w), jnp.float32),
            jax.ShapeDtypeStruct((b * h * w,), jnp.int32),
            jax.ShapeDtypeStruct((), jnp.float32),
            jax.ShapeDtypeStruct((), jnp.float32),
            jax.ShapeDtypeStruct((), jnp.float32),
            jax.ShapeDtypeStruct((), jnp.float32),
        ),
        out_specs=(
            pl.BlockSpec(memory_space=pltpu.VMEM),
            pl.BlockSpec(memory_space=pltpu.VMEM),
            smem_scalar, smem_scalar, smem_scalar, smem_scalar,
        ),
    )(z)

    return (q, (se, ae, aux, cl), indices_flat)


# speedup vs baseline: 48.8219x; 1.1201x over previous
import jax
import jax.numpy as jnp
from jax.experimental import pallas as pl
from jax.experimental.pallas import tpu as pltpu

_DIM = 14
_N_E = 16384
_TEMP = 0.01
_EPS = 1e-5
_BETA = 0.25
_ENTROPY_LOSS_RATIO = 0.1


def _kron7(pos_b, big_b, small_b, base, npix):
    sub_a = jax.lax.broadcasted_iota(jnp.int32, (8, npix), 0)
    a = None
    for d in range(3):
        bit = jnp.bitwise_and(jnp.right_shift(sub_a, d), 1) == 1
        dd = base + d
        f = jnp.where(bit == pos_b[dd:dd + 1, :],
                      big_b[dd:dd + 1, :], small_b[dd:dd + 1, :])
        a = f if a is None else a * f
    sub_b = jax.lax.broadcasted_iota(jnp.int32, (16, npix), 0)
    bb = None
    for d in range(4):
        bit = jnp.bitwise_and(jnp.right_shift(sub_b, d), 1) == 1
        dd = base + 3 + d
        f = jnp.where(bit == pos_b[dd:dd + 1, :],
                      big_b[dd:dd + 1, :], small_b[dd:dd + 1, :])
        bb = f if bb is None else bb * f
    return (bb[:, None, :] * a[None, :, :]).reshape(128, npix)


def _lfq_body(z_ref, sign_ref, idx_ref, se_ref, ae_ref, aux_ref, cl_ref):
    z4 = z_ref[...]
    bsz, _, hh, ww = z4.shape
    npix = hh * ww
    n_tok = bsz * npix

    pos4 = z4 > 0.0
    sgn4 = jnp.where(pos4, 1.0, -1.0).astype(jnp.float32)
    sign_ref[...] = sgn4

    z = z4.reshape(bsz, _DIM, npix)
    pos = z > 0.0

    d_iota = jax.lax.broadcasted_iota(jnp.int32, z.shape, 1)
    w = jnp.left_shift(jnp.int32(1), d_iota)
    idx_ref[...] = jnp.sum(jnp.where(pos, w, 0), axis=1).reshape(-1)

    scale = jnp.float32(4.0 / _TEMP)
    x = jnp.abs(z) * scale
    u = jnp.exp(-x)
    inv = 1.0 / (1.0 + u)
    big = inv
    small = u * inv

    hb = jnp.log1p(u) + x * small
    se = jnp.sum(hb) / jnp.float32(n_tok)

    cl = jnp.sum((sgn4 - z4) ** 2) / jnp.float32(n_tok * _DIM)

    def _dot_t(a, bb_):
        return jax.lax.dot_general(
            a, bb_, (((1,), (1,)), ((), ())),
            preferred_element_type=jnp.float32)

    acc = jnp.zeros((128, 128), jnp.float32)
    for b in range(bsz):
        lot = _kron7(pos[b], big[b], small[b], 0, npix)
        hit = _kron7(pos[b], big[b], small[b], 7, npix)
        lh = hit.astype(jnp.bfloat16)
        ll = (hit - lh.astype(jnp.float32)).astype(jnp.bfloat16)
        rh = lot.astype(jnp.bfloat16)
        rl = (lot - rh.astype(jnp.float32)).astype(jnp.bfloat16)
        acc = acc + (_dot_t(lh, rh) + (_dot_t(lh, rl) + _dot_t(ll, rh)))

    q = acc * (1.0 / jnp.float32(n_tok))
    ae = -jnp.sum(q * jnp.log(q + jnp.float32(_EPS)))

    se_ref[...] = se
    ae_ref[...] = ae
    aux_ref[...] = jnp.float32(_ENTROPY_LOSS_RATIO) * (se - ae)
    cl_ref[...] = jnp.float32(_BETA) * cl


def kernel(z, codebook):
    del codebook
    b, d, h, w = z.shape

    smem_scalar = pl.BlockSpec(memory_space=pltpu.SMEM)
    q, indices_flat, se, ae, aux, cl = pl.pallas_call(
        _lfq_body,
        out_shape=(
            jax.ShapeDtypeStruct((b, d, h, w), jnp.float32),
            jax.ShapeDtypeStruct((b * h * w,), jnp.int32),
            jax.ShapeDtypeStruct((), jnp.float32),
            jax.ShapeDtypeStruct((), jnp.float32),
            jax.ShapeDtypeStruct((), jnp.float32),
            jax.ShapeDtypeStruct((), jnp.float32),
        ),
        out_specs=(
            pl.BlockSpec(memory_space=pltpu.VMEM),
            pl.BlockSpec(memory_space=pltpu.VMEM),
            smem_scalar, smem_scalar, smem_scalar, smem_scalar,
        ),
    )(z)

    return (q, (se, ae, aux, cl), indices_flat)


# speedup vs baseline: 61.4532x; 1.2587x over previous
import jax
import jax.numpy as jnp
from jax.experimental import pallas as pl
from jax.experimental.pallas import tpu as pltpu

_DIM = 14
_N_E = 16384
_TEMP = 0.01
_EPS = 1e-5
_BETA = 0.25
_ENTROPY_LOSS_RATIO = 0.1


def _kron7(fp_b, fm_b, base, npix):
    sub_a = jax.lax.broadcasted_iota(jnp.int32, (8, 1), 0)
    a = None
    for d in range(3):
        bit = jnp.bitwise_and(jnp.right_shift(sub_a, d), 1) == 1
        dd = base + d
        f = jnp.where(bit, fp_b[dd:dd + 1, :], fm_b[dd:dd + 1, :])
        a = f if a is None else a * f
    sub_b = jax.lax.broadcasted_iota(jnp.int32, (16, 1), 0)
    bb = None
    for d in range(4):
        bit = jnp.bitwise_and(jnp.right_shift(sub_b, d), 1) == 1
        dd = base + 3 + d
        f = jnp.where(bit, fp_b[dd:dd + 1, :], fm_b[dd:dd + 1, :])
        bb = f if bb is None else bb * f
    return (bb[:, None, :] * a[None, :, :]).reshape(128, npix)


def _lfq_body(z_ref, sign_ref, idx_ref, se_ref, ae_ref, aux_ref, cl_ref):
    z4 = z_ref[...]
    bsz, _, hh, ww = z4.shape
    npix = hh * ww
    n_tok = bsz * npix

    pos4 = z4 > 0.0
    sgn4 = jnp.where(pos4, 1.0, -1.0).astype(jnp.float32)
    sign_ref[...] = sgn4

    z = z4.reshape(bsz, _DIM, npix)
    pos = z > 0.0

    d_iota = jax.lax.broadcasted_iota(jnp.int32, z.shape, 1)
    w = jnp.left_shift(jnp.int32(1), d_iota)
    idx_ref[...] = jnp.sum(jnp.where(pos, w, 0), axis=1).reshape(-1)

    scale = jnp.float32(4.0 / _TEMP)
    x = jnp.abs(z) * scale
    u = jnp.exp(-x)
    inv = 1.0 / (1.0 + u)
    big = inv
    small = u * inv

    hb = jnp.log1p(u) + x * small
    se = jnp.sum(hb) / jnp.float32(n_tok)

    cl = jnp.sum((sgn4 - z4) ** 2) / jnp.float32(n_tok * _DIM)

    def _dot_t(a, bb_):
        return jax.lax.dot_general(
            a, bb_, (((1,), (1,)), ((), ())),
            preferred_element_type=jnp.float32)

    fp = jnp.where(pos, big, small)
    fm = jnp.where(pos, small, big)
    acc = jnp.zeros((128, 128), jnp.float32)
    for b in range(bsz):
        lot = _kron7(fp[b], fm[b], 0, npix)
        hit = _kron7(fp[b], fm[b], 7, npix)
        lh = hit.astype(jnp.bfloat16)
        ll = (hit - lh.astype(jnp.float32)).astype(jnp.bfloat16)
        rh = lot.astype(jnp.bfloat16)
        rl = (lot - rh.astype(jnp.float32)).astype(jnp.bfloat16)
        acc = acc + (_dot_t(lh, rh) + (_dot_t(lh, rl) + _dot_t(ll, rh)))

    q = acc * (1.0 / jnp.float32(n_tok))
    ae = -jnp.sum(q * jnp.log(q + jnp.float32(_EPS)))

    se_ref[...] = se
    ae_ref[...] = ae
    aux_ref[...] = jnp.float32(_ENTROPY_LOSS_RATIO) * (se - ae)
    cl_ref[...] = jnp.float32(_BETA) * cl


def kernel(z, codebook):
    del codebook
    b, d, h, w = z.shape

    smem_scalar = pl.BlockSpec(memory_space=pltpu.SMEM)
    q, indices_flat, se, ae, aux, cl = pl.pallas_call(
        _lfq_body,
        out_shape=(
            jax.ShapeDtypeStruct((b, d, h, w), jnp.float32),
            jax.ShapeDtypeStruct((b * h * w,), jnp.int32),
            jax.ShapeDtypeStruct((), jnp.float32),
            jax.ShapeDtypeStruct((), jnp.float32),
            jax.ShapeDtypeStruct((), jnp.float32),
            jax.ShapeDtypeStruct((), jnp.float32),
        ),
        out_specs=(
            pl.BlockSpec(memory_space=pltpu.VMEM),
            pl.BlockSpec(memory_space=pltpu.VMEM),
            smem_scalar, smem_scalar, smem_scalar, smem_scalar,
        ),
    )(z)

    return (q, (se, ae, aux, cl), indices_flat)
